# baseline (speedup 1.0000x reference)
import jax
import jax.numpy as jnp
from jax import lax
from jax.experimental import pallas as pl
from jax.experimental.pallas import tpu as pltpu

_NB = 32


def _conv_relu(x, w_ref, b_ref, kh, kw, out_bf16):
    nb, h, _, w = x.shape
    oh, ow = h - kh + 1, w - kw + 1
    pieces = [x[:, i:i + oh, :, j:j + ow] for i in range(kh) for j in range(kw)]
    patches = jnp.concatenate(pieces, axis=2).astype(jnp.bfloat16)
    k = patches.shape[2]
    p2 = patches.reshape(nb * oh, k, ow)
    wmat = w_ref[...]
    wb = jnp.broadcast_to(wmat[None], (nb * oh,) + wmat.shape)
    y = lax.dot_general(wb, p2, dimension_numbers=(((2,), (1,)), ((0,), (0,))),
                        preferred_element_type=jnp.float32)
    y = y.reshape(nb, oh, wmat.shape[0], ow)
    y = jnp.maximum(y + b_ref[...][None, None], 0.0)
    return y.astype(jnp.bfloat16) if out_bf16 else y


def _max_pool(x, kp, w_stride):
    _, h, _, w = x.shape
    oh, ow1 = h - kp + 1, w - kp + 1
    m = x[:, :, :, 0:ow1]
    for j in range(1, kp):
        m = jnp.maximum(m, x[:, :, :, j:j + ow1])
    r = m[:, 0:oh]
    for i in range(1, kp):
        r = jnp.maximum(r, m[:, i:i + oh])
    if w_stride == 1:
        return r
    pw = (w - kp) // w_stride + 1
    rows = lax.broadcasted_iota(jnp.int32, (ow1, pw), 0)
    cols = lax.broadcasted_iota(jnp.int32, (ow1, pw), 1)
    sel = (rows == w_stride * cols).astype(x.dtype)
    out = lax.dot_general(r, sel, dimension_numbers=(((3,), (0,)), ((), ())),
                          preferred_element_type=jnp.float32)
    return out.astype(x.dtype)


def _textcnn_block_kernel(x_ref, w1_ref, b1_ref, w2_ref, b2_ref, w3_ref,
                          b3_ref, wf_ref, bf_ref, o_ref):
    x = x_ref[...]
    a = _max_pool(_conv_relu(x, w1_ref, b1_ref, 7, 3, True), 5, 3)
    a = _max_pool(_conv_relu(a, w2_ref, b2_ref, 5, 5, True), 3, 1)
    a = _max_pool(_conv_relu(a, w3_ref, b3_ref, 3, 3, True), 2, 1)
    nb = a.shape[0]
    af = a.reshape(nb, 21 * 32 * 23)
    o = lax.dot_general(af, wf_ref[...],
                        dimension_numbers=(((1,), (0,)), ((), ())),
                        preferred_element_type=jnp.float32)
    o_ref[0] = o + bf_ref[...]


def _weight_spec(shape):
    return pl.BlockSpec(shape, lambda *_: (0,) * len(shape))


def kernel(x, w1, b1, w2, b2, w3, b3, wf, bf):
    x = x.reshape(-1, 40, 300).astype(jnp.float32)
    b = x.shape[0]
    xph = jnp.transpose(x.reshape(b, 40, 100, 3), (0, 1, 3, 2))
    w1s = w1.reshape(8, 21, 8)[:, :, :3].reshape(8, 63)
    wff = jnp.transpose(wf.reshape(5, 21 * 32 * 23)).astype(jnp.bfloat16)

    out = pl.pallas_call(
        _textcnn_block_kernel,
        out_shape=jax.ShapeDtypeStruct((b // _NB, _NB, 5), jnp.float32),
        grid=(b // _NB,),
        in_specs=[
            pl.BlockSpec((_NB, 40, 3, 100), lambda i: (i, 0, 0, 0)),
            _weight_spec((8, 63)), _weight_spec((8, 1)),
            _weight_spec((16, 200)), _weight_spec((16, 1)),
            _weight_spec((32, 144)), _weight_spec((32, 1)),
            _weight_spec((21 * 32 * 23, 5)), _weight_spec((1, 5)),
        ],
        out_specs=pl.BlockSpec((1, _NB, 5), lambda i: (i, 0, 0)),
        compiler_params=pltpu.CompilerParams(
            dimension_semantics=("parallel",),
            vmem_limit_bytes=100 * 1024 * 1024),
    )(xph, w1s, b1, w2, b2, w3, b3, wff, bf)
    return out.reshape(b, 5)


# speedup vs baseline: 1.5164x; 1.5164x over previous
import jax
import jax.numpy as jnp
from jax import lax
from jax.experimental import pallas as pl
from jax.experimental.pallas import tpu as pltpu

_NB = 32


def _conv_relu(x, w_ref, b_ref, kh, kw, out_bf16):
    nb, h, _, w = x.shape
    oh, ow = h - kh + 1, w - kw + 1
    pieces = [x[:, i:i + oh, :, j:j + ow] for i in range(kh) for j in range(kw)]
    patches = jnp.concatenate(pieces, axis=2).astype(jnp.bfloat16)
    k = patches.shape[2]
    p2 = patches.reshape(nb * oh, k, ow)
    wmat = w_ref[...]
    wb = jnp.broadcast_to(wmat[None], (nb * oh,) + wmat.shape)
    y = lax.dot_general(wb, p2, dimension_numbers=(((2,), (1,)), ((0,), (0,))),
                        preferred_element_type=jnp.float32)
    y = y.reshape(nb, oh, wmat.shape[0], ow)
    y = jnp.maximum(y + b_ref[...][None, None], 0.0)
    return y.astype(jnp.bfloat16) if out_bf16 else y


def _max_pool(x, kp, w_stride):
    _, h, _, w = x.shape
    oh, ow1 = h - kp + 1, w - kp + 1
    m = x[:, :, :, 0:ow1]
    for j in range(1, kp):
        m = jnp.maximum(m, x[:, :, :, j:j + ow1])
    r = m[:, 0:oh]
    for i in range(1, kp):
        r = jnp.maximum(r, m[:, i:i + oh])
    if w_stride == 1:
        return r
    pw = (w - kp) // w_stride + 1
    rows = lax.broadcasted_iota(jnp.int32, (ow1, pw), 0)
    cols = lax.broadcasted_iota(jnp.int32, (ow1, pw), 1)
    sel = (rows == w_stride * cols).astype(x.dtype)
    out = lax.dot_general(r, sel, dimension_numbers=(((3,), (0,)), ((), ())),
                          preferred_element_type=jnp.float32)
    return out.astype(x.dtype)


def _textcnn_block_kernel(x_ref, w1_ref, b1_ref, w2_ref, b2_ref, w3_ref,
                          b3_ref, wf_ref, bf_ref, o_ref):
    x = x_ref[...]
    a = _max_pool(_conv_relu(x, w1_ref, b1_ref, 7, 3, True), 5, 3)
    a = _max_pool(_conv_relu(a, w2_ref, b2_ref, 5, 5, True), 3, 1)
    a = _max_pool(_conv_relu(a, w3_ref, b3_ref, 3, 3, True), 2, 1)
    nb = a.shape[0]
    af = a.reshape(nb, 21 * 32 * 23)
    o = lax.dot_general(af, wf_ref[...],
                        dimension_numbers=(((1,), (0,)), ((), ())),
                        preferred_element_type=jnp.float32)
    o_ref[0] = o + bf_ref[...]


def _weight_spec(shape):
    return pl.BlockSpec(shape, lambda *_: (0,) * len(shape))


def kernel(x, w1, b1, w2, b2, w3, b3, wf, bf):
    x = x.reshape(-1, 40, 300).astype(jnp.float32)
    b = x.shape[0]
    xph = jnp.transpose(x.reshape(b, 40, 100, 3), (0, 1, 3, 2))
    xph = jnp.pad(xph, ((0, 0), (0, 0), (0, 1), (0, 0)))
    w1s = w1.reshape(8, 21, 8)[:, :, :4].reshape(8, 84)
    wff = jnp.transpose(wf.reshape(5, 21 * 32 * 23)).astype(jnp.bfloat16)

    out = pl.pallas_call(
        _textcnn_block_kernel,
        out_shape=jax.ShapeDtypeStruct((b // _NB, _NB, 5), jnp.float32),
        grid=(b // _NB,),
        in_specs=[
            pl.BlockSpec((_NB, 40, 4, 100), lambda i: (i, 0, 0, 0)),
            _weight_spec((8, 84)), _weight_spec((8, 1)),
            _weight_spec((16, 200)), _weight_spec((16, 1)),
            _weight_spec((32, 144)), _weight_spec((32, 1)),
            _weight_spec((21 * 32 * 23, 5)), _weight_spec((1, 5)),
        ],
        out_specs=pl.BlockSpec((1, _NB, 5), lambda i: (i, 0, 0)),
        compiler_params=pltpu.CompilerParams(
            dimension_semantics=("parallel",),
            vmem_limit_bytes=100 * 1024 * 1024),
    )(xph, w1s, b1, w2, b2, w3, b3, wff, bf)
    return out.reshape(b, 5)
